# baseline (speedup 1.0000x reference)
import jax
import jax.numpy as jnp
from jax.experimental import pallas as pl
from jax.experimental.pallas import tpu as pltpu

B, S, D, F, T = 2, 4096, 1024, 4096, 4
M = B * S

TFP = 512
NFP = F // TFP
TMP = M // NFP

TM = 2048
TF = 1024
NM = M // TM
NF = F // TF


def _prep_kernel(tid_ref, win_ref, dwi_ref, wout_ref, dwo_ref,
                 bin_ref, dbi_ref, bout_ref, dbo_ref, x_ref,
                 w1_out, w2_out, b1_out, b2_out, xb_out):
    w1_out[...] = (win_ref[...] + dwi_ref[0]).astype(jnp.bfloat16)
    w2_out[...] = (wout_ref[...] + dwo_ref[0]).astype(jnp.bfloat16)
    b1_out[...] = bin_ref[...] + dbi_ref[0]
    b2_out[...] = bout_ref[...] + dbo_ref[0]
    xb_out[...] = x_ref[...].astype(jnp.bfloat16)


def _ffn_kernel(xb_ref, w1_ref, b1_ref, w2_ref, b2_ref, out_ref):
    f = pl.program_id(1)
    h = jax.lax.dot_general(
        xb_ref[...], w1_ref[...], (((1,), (1,)), ((), ())),
        preferred_element_type=jnp.float32)
    h = jax.nn.gelu(h + b1_ref[...])
    acc = jax.lax.dot_general(
        h.astype(jnp.bfloat16), w2_ref[...], (((1,), (1,)), ((), ())),
        preferred_element_type=jnp.float32)

    @pl.when(f == 0)
    def _init():
        out_ref[...] = acc + b2_ref[...]

    @pl.when(f != 0)
    def _acc():
        out_ref[...] += acc


def kernel(x, W_in, b_in, W_out, b_out, dW_in, db_in, dW_out, db_out, task_id):
    xm = x.reshape(M, D)
    b_in2 = b_in.reshape(1, F)
    b_out2 = b_out.reshape(1, D)
    db_in3 = db_in.reshape(T, 1, F)
    db_out3 = db_out.reshape(T, 1, D)
    tid = jnp.asarray(task_id, jnp.int32).reshape(1)

    prep_spec = pltpu.PrefetchScalarGridSpec(
        num_scalar_prefetch=1,
        grid=(NFP,),
        in_specs=[
            pl.BlockSpec((TFP, D), lambda f, t: (f, 0)),
            pl.BlockSpec((1, TFP, D), lambda f, t: (t[0], f, 0)),
            pl.BlockSpec((D, TFP), lambda f, t: (0, f)),
            pl.BlockSpec((1, D, TFP), lambda f, t: (t[0], 0, f)),
            pl.BlockSpec((1, TFP), lambda f, t: (0, f)),
            pl.BlockSpec((1, 1, TFP), lambda f, t: (t[0], 0, f)),
            pl.BlockSpec((1, D), lambda f, t: (0, 0)),
            pl.BlockSpec((1, 1, D), lambda f, t: (t[0], 0, 0)),
            pl.BlockSpec((TMP, D), lambda f, t: (f, 0)),
        ],
        out_specs=[
            pl.BlockSpec((TFP, D), lambda f, t: (f, 0)),
            pl.BlockSpec((D, TFP), lambda f, t: (0, f)),
            pl.BlockSpec((1, TFP), lambda f, t: (0, f)),
            pl.BlockSpec((1, D), lambda f, t: (0, 0)),
            pl.BlockSpec((TMP, D), lambda f, t: (f, 0)),
        ],
    )
    w1e, w2e, b1e, b2e, xb = pl.pallas_call(
        _prep_kernel,
        grid_spec=prep_spec,
        out_shape=[
            jax.ShapeDtypeStruct((F, D), jnp.bfloat16),
            jax.ShapeDtypeStruct((D, F), jnp.bfloat16),
            jax.ShapeDtypeStruct((1, F), jnp.float32),
            jax.ShapeDtypeStruct((1, D), jnp.float32),
            jax.ShapeDtypeStruct((M, D), jnp.bfloat16),
        ],
    )(tid, W_in, dW_in, W_out, dW_out, b_in2, db_in3, b_out2, db_out3, xm)

    out = pl.pallas_call(
        _ffn_kernel,
        grid=(NM, NF),
        in_specs=[
            pl.BlockSpec((TM, D), lambda m, f: (m, 0)),
            pl.BlockSpec((TF, D), lambda m, f: (f, 0)),
            pl.BlockSpec((1, TF), lambda m, f: (0, f)),
            pl.BlockSpec((D, TF), lambda m, f: (0, f)),
            pl.BlockSpec((1, D), lambda m, f: (0, 0)),
        ],
        out_specs=pl.BlockSpec((TM, D), lambda m, f: (m, 0)),
        out_shape=jax.ShapeDtypeStruct((M, D), jnp.float32),
        compiler_params=pltpu.CompilerParams(
            dimension_semantics=("parallel", "arbitrary")),
    )(xb, w1e, b1e, w2e, b2e)
    return out.reshape(B, S, D)


# speedup vs baseline: 1.2029x; 1.2029x over previous
import jax
import jax.numpy as jnp
from jax.experimental import pallas as pl
from jax.experimental.pallas import tpu as pltpu

B, S, D, F, T = 2, 4096, 1024, 4096, 4
M = B * S

TFP = 512
NFP = F // TFP

TM = 1024
NM = M // TM


def _prep_kernel(tid_ref, win_ref, dwi_ref, wout_ref, dwo_ref,
                 bin_ref, dbi_ref, bout_ref, dbo_ref,
                 w1_out, w2_out, b1_out, b2_out):
    w1_out[...] = (win_ref[...] + dwi_ref[0]).astype(jnp.bfloat16)
    w2_out[...] = (wout_ref[...] + dwo_ref[0]).astype(jnp.bfloat16)
    b1_out[...] = bin_ref[...] + dbi_ref[0]
    b2_out[...] = bout_ref[...] + dbo_ref[0]


def _ffn_kernel(xb_ref, w1_ref, b1_ref, w2_ref, b2_ref, out_ref):
    xb = xb_ref[...].astype(jnp.bfloat16)
    h = jax.lax.dot_general(
        xb, w1_ref[...], (((1,), (1,)), ((), ())),
        preferred_element_type=jnp.float32)
    h = jax.nn.gelu(h + b1_ref[...]).astype(jnp.bfloat16)
    out_ref[...] = jax.lax.dot_general(
        h, w2_ref[...], (((1,), (1,)), ((), ())),
        preferred_element_type=jnp.float32) + b2_ref[...]


def kernel(x, W_in, b_in, W_out, b_out, dW_in, db_in, dW_out, db_out, task_id):
    xm = x.reshape(M, D)
    b_in2 = b_in.reshape(1, F)
    b_out2 = b_out.reshape(1, D)
    db_in3 = db_in.reshape(T, 1, F)
    db_out3 = db_out.reshape(T, 1, D)
    tid = jnp.asarray(task_id, jnp.int32).reshape(1)

    prep_spec = pltpu.PrefetchScalarGridSpec(
        num_scalar_prefetch=1,
        grid=(NFP,),
        in_specs=[
            pl.BlockSpec((TFP, D), lambda f, t: (f, 0)),
            pl.BlockSpec((1, TFP, D), lambda f, t: (t[0], f, 0)),
            pl.BlockSpec((D, TFP), lambda f, t: (0, f)),
            pl.BlockSpec((1, D, TFP), lambda f, t: (t[0], 0, f)),
            pl.BlockSpec((1, TFP), lambda f, t: (0, f)),
            pl.BlockSpec((1, 1, TFP), lambda f, t: (t[0], 0, f)),
            pl.BlockSpec((1, D), lambda f, t: (0, 0)),
            pl.BlockSpec((1, 1, D), lambda f, t: (t[0], 0, 0)),
        ],
        out_specs=[
            pl.BlockSpec((TFP, D), lambda f, t: (f, 0)),
            pl.BlockSpec((D, TFP), lambda f, t: (0, f)),
            pl.BlockSpec((1, TFP), lambda f, t: (0, f)),
            pl.BlockSpec((1, D), lambda f, t: (0, 0)),
        ],
    )
    w1e, w2e, b1e, b2e = pl.pallas_call(
        _prep_kernel,
        grid_spec=prep_spec,
        out_shape=[
            jax.ShapeDtypeStruct((F, D), jnp.bfloat16),
            jax.ShapeDtypeStruct((D, F), jnp.bfloat16),
            jax.ShapeDtypeStruct((1, F), jnp.float32),
            jax.ShapeDtypeStruct((1, D), jnp.float32),
        ],
    )(tid, W_in, dW_in, W_out, dW_out, b_in2, db_in3, b_out2, db_out3)

    out = pl.pallas_call(
        _ffn_kernel,
        grid=(NM,),
        in_specs=[
            pl.BlockSpec((TM, D), lambda m: (m, 0)),
            pl.BlockSpec((F, D), lambda m: (0, 0)),
            pl.BlockSpec((1, F), lambda m: (0, 0)),
            pl.BlockSpec((D, F), lambda m: (0, 0)),
            pl.BlockSpec((1, D), lambda m: (0, 0)),
        ],
        out_specs=pl.BlockSpec((TM, D), lambda m: (m, 0)),
        out_shape=jax.ShapeDtypeStruct((M, D), jnp.float32),
        compiler_params=pltpu.CompilerParams(
            dimension_semantics=("arbitrary",)),
    )(xm, w1e, b1e, w2e, b2e)
    return out.reshape(B, S, D)
